# Initial kernel scaffold; baseline (speedup 1.0000x reference)
import jax
import jax.numpy as jnp
from jax.experimental import pallas as pl


def kernel(dense_inputs, sparse_inputs, tables, w_wide, b_wide, W1, b1, W2, b2, W3, b3, W4, b4):
    raise NotImplementedError("write your pallas kernel here")


# speedup vs baseline: 7.7068x; 7.7068x over previous
import functools

import jax
import jax.numpy as jnp
from jax import lax
from jax.experimental import pallas as pl
from jax.experimental.pallas import tpu as pltpu
from jax.experimental.pallas import tpu_sc as plsc

N_FIELDS = 26
VOCAB = 100000
EMBED_DIM = 16
N_DENSE = 13
BATCH = 16384

_NC = 2
_NS = 16
_NW = _NC * _NS
_ROWS = BATCH * N_FIELDS
_ROWS_W = _ROWS // _NW
_CH = 1664
_NCH = _ROWS_W // _CH
_PERIOD = 208


def _sc_gather(tables_flat, idx_flat, offs):
    mesh = plsc.VectorSubcoreMesh(core_axis_name="c", subcore_axis_name="s")

    @functools.partial(
        pl.kernel,
        mesh=mesh,
        compiler_params=pltpu.CompilerParams(use_tc_tiling_on_sc=False),
        out_type=jax.ShapeDtypeStruct((_ROWS, EMBED_DIM), jnp.float32),
        scratch_types=[
            pltpu.VMEM((_CH,), jnp.int32),
            pltpu.VMEM((_CH,), jnp.int32),
            pltpu.VMEM((_CH, EMBED_DIM), jnp.float32),
            pltpu.VMEM((_PERIOD,), jnp.int32),
            pltpu.SemaphoreType.DMA,
        ],
    )
    def k(tab_hbm, idx_hbm, off_hbm, out_hbm, raw_v, idx_v, rows_v, off_v, sem):
        wid = lax.axis_index("s") * _NC + lax.axis_index("c")
        pltpu.sync_copy(off_hbm, off_v)
        base_w = wid * _ROWS_W

        def chunk(c, carry):
            base = base_w + c * _CH
            pltpu.sync_copy(idx_hbm.at[pl.ds(base, _CH)], raw_v)

            def grp(m, carry2):
                s = m * _PERIOD
                for j in range(_PERIOD // 16):
                    sl = pl.ds(s + j * 16, 16)
                    idx_v[sl] = raw_v[sl] + off_v[j * 16:(j + 1) * 16]
                return carry2

            lax.fori_loop(0, _CH // _PERIOD, grp, 0)
            pltpu.async_copy(tab_hbm.at[idx_v], rows_v, sem).wait()
            pltpu.sync_copy(rows_v, out_hbm.at[pl.ds(base, _CH)])
            return carry

        lax.fori_loop(0, _NCH, chunk, 0)

    return k(tables_flat, idx_flat, offs)


_BB = 1024


def _mlp_body(dense_ref, emb_ref, w1d_ref, w1e_ref, b1_ref, w2_ref, b2_ref,
              w3_ref, b3_ref, w4r_ref, wwr_ref, bsum_ref, out_ref):
    dense = dense_ref[...]
    emb = emb_ref[...]
    h = jnp.dot(dense, w1d_ref[...], preferred_element_type=jnp.float32)
    h = h + jnp.dot(emb, w1e_ref[...], preferred_element_type=jnp.float32)
    h = jnp.maximum(h + b1_ref[...], 0.0)
    h = jnp.maximum(jnp.dot(h, w2_ref[...], preferred_element_type=jnp.float32)
                    + b2_ref[...], 0.0)
    h = jnp.maximum(jnp.dot(h, w3_ref[...], preferred_element_type=jnp.float32)
                    + b3_ref[...], 0.0)
    deep = jnp.sum(h * w4r_ref[...], axis=1, keepdims=True)
    wide = jnp.sum(dense * wwr_ref[...], axis=1, keepdims=True)
    z = 0.5 * (wide + deep) + bsum_ref[...]
    out_ref[...] = 1.0 / (1.0 + jnp.exp(-z))


def _mlp(dense, emb, w1d, w1e, b1, W2, b2, W3, b3, w4r, wwr, bsum):
    grid = (BATCH // _BB,)
    full = lambda shape: pl.BlockSpec(shape, lambda i: (0, 0))
    return pl.pallas_call(
        _mlp_body,
        grid=grid,
        in_specs=[
            pl.BlockSpec((_BB, N_DENSE), lambda i: (i, 0)),
            pl.BlockSpec((_BB, N_FIELDS * EMBED_DIM), lambda i: (i, 0)),
            full(w1d.shape), full(w1e.shape), full(b1.shape),
            full(W2.shape), full(b2.shape),
            full(W3.shape), full(b3.shape),
            full(w4r.shape), full(wwr.shape), full(bsum.shape),
        ],
        out_specs=pl.BlockSpec((_BB, 1), lambda i: (i, 0)),
        out_shape=jax.ShapeDtypeStruct((BATCH, 1), jnp.float32),
    )(dense, emb, w1d, w1e, b1, W2, b2, W3, b3, w4r, wwr, bsum)


def kernel(dense_inputs, sparse_inputs, tables, w_wide, b_wide,
           W1, b1, W2, b2, W3, b3, W4, b4):
    tables_flat = tables.reshape(N_FIELDS * VOCAB, EMBED_DIM)
    idx_flat = sparse_inputs.astype(jnp.int32).reshape(_ROWS)
    offs = jnp.tile(jnp.arange(N_FIELDS, dtype=jnp.int32) * VOCAB,
                    _PERIOD // N_FIELDS)

    emb_flat = _sc_gather(tables_flat, idx_flat, offs)
    emb = emb_flat.reshape(BATCH, N_FIELDS * EMBED_DIM)

    w1d = W1[:N_DENSE]
    w1e = W1[N_DENSE:]
    w4r = W4.reshape(1, -1)
    wwr = w_wide.reshape(1, -1)
    bsum = (0.5 * (b_wide + b4)).reshape(1, 1)

    return _mlp(dense_inputs, emb, w1d, w1e, b1.reshape(1, -1),
                W2, b2.reshape(1, -1), W3, b3.reshape(1, -1), w4r, wwr, bsum)
